# baseline (speedup 1.0000x reference)
import jax
import jax.numpy as jnp
import numpy as np
from jax.experimental import pallas as pl

INDEX_SHIFT = [[0, 0], [-1, 0], [1, 0], [0, 1], [-1, 1], [1, 1], [0, -1], [-1, -1], [1, -1]]
NY, NX = 496, 432
E = 32
NHEADS = 2


def _proj_body(x_ref, w_ref, b_ref, o_ref):
    o_ref[...] = (
        jnp.dot(x_ref[...], w_ref[...].T, preferred_element_type=jnp.float32)
        + b_ref[...]
    )


def _proj(x, W, b):
    N = x.shape[0]
    O = W.shape[0]
    BN = 4000
    assert N % BN == 0, N
    return pl.pallas_call(
        _proj_body,
        grid=(N // BN,),
        in_specs=[
            pl.BlockSpec((BN, x.shape[1]), lambda i: (i, 0)),
            pl.BlockSpec(W.shape, lambda i: (0, 0)),
            pl.BlockSpec((1, O), lambda i: (0, 0)),
        ],
        out_specs=pl.BlockSpec((BN, O), lambda i: (i, 0)),
        out_shape=jax.ShapeDtypeStruct((N, O), jnp.float32),
    )(x, W, jnp.reshape(b, (1, O)))


def _mha_post(q_feat, key_t, W_in, b_in, W_out, b_out):
    hd = E // NHEADS
    qkv = _proj(q_feat, W_in[:E], b_in[:E])
    N = q_feat.shape[0]
    k = _proj(key_t.reshape(N * 9, E), W_in[E:2 * E], b_in[E:2 * E]).reshape(N, 9, E)
    v = _proj(key_t.reshape(N * 9, E), W_in[2 * E:], b_in[2 * E:]).reshape(N, 9, E)
    q = qkv.reshape(N, NHEADS, hd)
    kh = k.reshape(N, 9, NHEADS, hd).transpose(0, 2, 1, 3)
    vh = v.reshape(N, 9, NHEADS, hd).transpose(0, 2, 1, 3)
    scores = jnp.einsum("nhd,nhjd->nhj", q, kh) / float(np.sqrt(hd))
    attn = jax.nn.softmax(scores, axis=-1)
    out = jnp.einsum("nhj,nhjd->nhd", attn, vh).reshape(N, E)
    return _proj(out, W_out, b_out)


def _cross(q_feat, q_coor, kv_feat, kv_coor, W_in, b_in, W_out, b_out):
    M = kv_feat.shape[0]
    kv_lin = kv_coor[:, 0] * NX + kv_coor[:, 1]
    table = jnp.full((NY * NX,), -1, dtype=jnp.int32).at[kv_lin].set(
        jnp.arange(M, dtype=jnp.int32))
    keys = []
    for dy, dx in INDEX_SHIFT:
        sy = jnp.clip(q_coor[:, 0] + dy, 0, NY - 1)
        sx = jnp.clip(q_coor[:, 1] + dx, 0, NX - 1)
        sel = table[sy * NX + sx]
        mask = sel >= 0
        tmp = jnp.where(mask[:, None], jnp.take(kv_feat, jnp.clip(sel, 0, M - 1), axis=0), 0.0)
        keys.append(tmp)
    key_t = jnp.stack(keys, axis=0).transpose(1, 0, 2)
    out = _mha_post(q_feat, key_t, W_in, b_in, W_out, b_out)
    out_feat = q_feat + out
    idx = q_coor[:, 0] * NX + q_coor[:, 1]
    canvas = jnp.zeros((E, NY * NX), dtype=q_feat.dtype).at[:, idx].set(out_feat.T)
    return canvas.reshape(1, E, NY, NX)


def kernel(li_bev_feats, li_bev_coors, ra_bev_feats, ra_bev_coors, W_in1, b_in1, W_out1, b_out1, W_in2, b_in2, W_out2, b_out2):
    li_lst, ra_lst = [], []
    for i in range(li_bev_feats.shape[0]):
        li_lst.append(_cross(li_bev_feats[i], li_bev_coors[i], ra_bev_feats[i], ra_bev_coors[i], W_in1, b_in1, W_out1, b_out1))
        ra_lst.append(_cross(ra_bev_feats[i], ra_bev_coors[i], li_bev_feats[i], li_bev_coors[i], W_in2, b_in2, W_out2, b_out2))
    return jnp.concatenate(li_lst, axis=0), jnp.concatenate(ra_lst, axis=0)
